# Initial kernel scaffold; baseline (speedup 1.0000x reference)
import jax
import jax.numpy as jnp
from jax.experimental import pallas as pl


def kernel(h_neigh, h_self, edge_index, edge_features, W_self, W_neigh):
    raise NotImplementedError("write your pallas kernel here")


# speedup vs baseline: 3.2631x; 3.2631x over previous
import functools

import jax
import jax.numpy as jnp
from jax import lax
from jax.experimental import pallas as pl
from jax.experimental.pallas import tpu as pltpu
from jax.experimental.pallas import tpu_sc as plsc

_NC = 2
_NS = 16
_CH = 80
_NP = 10240


@functools.lru_cache(maxsize=None)
def _build_sc(N, E, DF, DE):
    R = E // _CH
    K = R // (_NC * _NS)
    assert R % (_NC * _NS) == 0
    stripe = _NP // _NS
    mesh = plsc.VectorSubcoreMesh(core_axis_name="c", subcore_axis_name="s",
                                  num_cores=_NC, num_subcores=_NS)

    @functools.partial(
        pl.kernel,
        out_type=(
            jax.ShapeDtypeStruct((_NC * _NP, DF), jnp.float32),
            jax.ShapeDtypeStruct((_NC * _NP, DE), jnp.float32),
            jax.ShapeDtypeStruct((_NC * _NP, DE), jnp.float32),
        ),
        mesh=mesh,
        compiler_params=pltpu.CompilerParams(use_tc_tiling_on_sc=False),
        scratch_types=[
            pltpu.VMEM_SHARED((_NP, DF), jnp.float32),
            pltpu.VMEM_SHARED((_NP, DE), jnp.float32),
            pltpu.VMEM_SHARED((_NP, DE), jnp.float32),
            pltpu.VMEM((_CH,), jnp.int32),
            pltpu.VMEM((_CH,), jnp.int32),
            pltpu.VMEM((_CH,), jnp.int32),
            pltpu.VMEM((_CH, DE), jnp.float32),
            pltpu.VMEM((_CH, DE), jnp.float32),
            pltpu.VMEM((_CH, DF), jnp.float32),
            pltpu.SemaphoreType.DMA,
            pltpu.SemaphoreType.DMA,
        ],
    )
    def sc_k(h_hbm, src_hbm, dst_hbm, ef_hbm, z_f_hbm, z_e_hbm,
             out_h, out_e, out_d,
             acc_h, acc_e, acc_d, src_v, dst_v, eidx_v, ef_v, ones_v,
             rows_v, sem, sem2):
        c = lax.axis_index("c")
        s = lax.axis_index("s")
        wid = s * _NC + c
        sb = s * stripe
        for t in range(stripe // _CH):
            o = sb + t * _CH
            pltpu.sync_copy(z_f_hbm.at[pl.ds(o, _CH)], acc_h.at[pl.ds(o, _CH)])
        pltpu.sync_copy(z_e_hbm.at[pl.ds(sb, stripe)], acc_e.at[pl.ds(sb, stripe)])
        pltpu.sync_copy(z_e_hbm.at[pl.ds(sb, stripe)], acc_d.at[pl.ds(sb, stripe)])
        one16 = jnp.ones((16,), jnp.float32)
        for r in range(_CH):
            ones_v[r] = one16
        plsc.subcore_barrier()

        @pl.loop(0, K)
        def _(k):
            base = (wid * K + k) * _CH
            pltpu.sync_copy(src_hbm.at[pl.ds(base, _CH)], src_v)
            pltpu.sync_copy(dst_hbm.at[pl.ds(base, _CH)], dst_v)
            for j in range(_CH // 16):
                eidx_v[pl.ds(16 * j, 16)] = lax.iota(jnp.int32, 16) + base + 16 * j
            pltpu.async_copy(ef_hbm.at[eidx_v], ef_v, sem2).wait()
            pltpu.async_copy(h_hbm.at[src_v], rows_v, sem).wait()
            pltpu.sync_copy(rows_v, acc_h.at[dst_v], add=True)
            pltpu.sync_copy(ef_v, acc_e.at[dst_v], add=True)
            pltpu.sync_copy(ones_v, acc_d.at[dst_v], add=True)

        plsc.subcore_barrier()
        ob = c * _NP + sb
        for t in range(stripe // _CH):
            pltpu.sync_copy(acc_h.at[pl.ds(sb + t * _CH, _CH)],
                            out_h.at[pl.ds(ob + t * _CH, _CH)])
        pltpu.sync_copy(acc_e.at[pl.ds(sb, stripe)], out_e.at[pl.ds(ob, stripe)])
        pltpu.sync_copy(acc_d.at[pl.ds(sb, stripe)], out_d.at[pl.ds(ob, stripe)])

    return sc_k


def _tc_body(hs_ref, ah0_ref, ah1_ref, ae0_ref, ae1_ref, ad0_ref, ad1_ref,
             wsT_ref, wn1T_ref, wn2T_ref, o_ref):
    deg = ad0_ref[:, 0:1] + ad1_ref[:, 0:1]
    inv = 1.0 / jnp.where(deg == 0.0, 1.0, deg)
    nm = (ah0_ref[...] + ah1_ref[...]) * inv
    em = (ae0_ref[...] + ae1_ref[...]) * inv
    hi = jax.lax.Precision.HIGHEST
    z = (jnp.dot(hs_ref[...], wsT_ref[...], precision=hi)
         + jnp.dot(nm, wn1T_ref[...], precision=hi)
         + jnp.dot(em, wn2T_ref[...], precision=hi))
    z = jnp.maximum(z, 0.0)
    nrm = jnp.sqrt(jnp.sum(z * z, axis=1, keepdims=True))
    o_ref[...] = z / jnp.where(nrm == 0.0, 1.0, nrm)


@functools.lru_cache(maxsize=None)
def _build_tc(N, DF, DE, DO, blk):
    g = N // blk
    ob = _NP // blk

    def spec(d, off):
        return pl.BlockSpec((blk, d), lambda i, o=off: (i + o, 0))

    full = lambda a, b: pl.BlockSpec((a, b), lambda i: (0, 0))
    return pl.pallas_call(
        _tc_body,
        grid=(g,),
        in_specs=[
            pl.BlockSpec((blk, DF), lambda i: (i, 0)),
            spec(DF, 0), spec(DF, ob),
            spec(DE, 0), spec(DE, ob),
            spec(DE, 0), spec(DE, ob),
            full(DF, DO), full(DF, DO), full(DE, DO),
        ],
        out_specs=pl.BlockSpec((blk, DO), lambda i: (i, 0)),
        out_shape=jax.ShapeDtypeStruct((N, DO), jnp.float32),
    )


def kernel(h_neigh, h_self, edge_index, edge_features, W_self, W_neigh):
    N, DF = h_neigh.shape
    E = edge_index.shape[1]
    DE = edge_features.shape[1]
    DO = W_self.shape[0]
    src = edge_index[0]
    dst = edge_index[1]
    z_f = jnp.zeros((_NP, DF), jnp.float32)
    z_e = jnp.zeros((_NP, DE), jnp.float32)
    out_h, out_e, out_d = _build_sc(N, E, DF, DE)(
        h_neigh, src, dst, edge_features, z_f, z_e)
    wsT = W_self.T
    wn1T = W_neigh[:, :DF].T
    wn2T = W_neigh[:, DF:].T
    return _build_tc(N, DF, DE, DO, 80)(
        h_self, out_h, out_h, out_e, out_e, out_d, out_d, wsT, wn1T, wn2T)


# speedup vs baseline: 4.5668x; 1.3995x over previous
import functools

import jax
import jax.numpy as jnp
from jax import lax
from jax.experimental import pallas as pl
from jax.experimental.pallas import tpu as pltpu
from jax.experimental.pallas import tpu_sc as plsc

_NC = 2
_NS = 16
_CH = 80
_SEG = 25
_NP = 10240


@functools.lru_cache(maxsize=None)
def _build_sc(N, E, DF, DE):
    R = E // _CH
    K = R // (_NC * _NS)
    assert R % (_NC * _NS) == 0
    stripe = _NP // _NS
    mesh = plsc.VectorSubcoreMesh(core_axis_name="c", subcore_axis_name="s",
                                  num_cores=_NC, num_subcores=_NS)

    @functools.partial(
        pl.kernel,
        out_type=(
            jax.ShapeDtypeStruct((_NC * _NP, DF), jnp.float32),
            jax.ShapeDtypeStruct((_NC * _NP, DE), jnp.float32),
            jax.ShapeDtypeStruct((_NC * _NP, DE), jnp.float32),
        ),
        mesh=mesh,
        compiler_params=pltpu.CompilerParams(use_tc_tiling_on_sc=False),
        scratch_types=[
            pltpu.VMEM_SHARED((_NP, DF), jnp.float32),
            pltpu.VMEM_SHARED((_NP, DE), jnp.float32),
            pltpu.VMEM_SHARED((_NP, DE), jnp.float32),
            pltpu.VMEM((_SEG, _CH), jnp.int32),
            pltpu.VMEM((_SEG, _CH), jnp.int32),
            pltpu.VMEM((_CH,), jnp.int32),
            pltpu.VMEM((_CH, DE), jnp.float32),
            pltpu.VMEM((_CH, DE), jnp.float32),
            pltpu.VMEM((_CH, DF), jnp.float32),
            pltpu.SemaphoreType.DMA,
            pltpu.SemaphoreType.DMA,
            pltpu.SemaphoreType.DMA,
        ],
    )
    def sc_k(h_hbm, src_hbm, dst_hbm, ef_hbm, z_f_hbm, z_e_hbm,
             out_h, out_e, out_d,
             acc_h, acc_e, acc_d, src_v, dst_v, eidx_v, ef_v, ones_v,
             rows_v, sem, sem2, sem3):
        c = lax.axis_index("c")
        s = lax.axis_index("s")
        wid = s * _NC + c
        sb = s * stripe
        for t in range(stripe // _CH):
            o = sb + t * _CH
            pltpu.sync_copy(z_f_hbm.at[pl.ds(o, _CH)], acc_h.at[pl.ds(o, _CH)])
        pltpu.sync_copy(z_e_hbm.at[pl.ds(sb, stripe)], acc_e.at[pl.ds(sb, stripe)])
        pltpu.sync_copy(z_e_hbm.at[pl.ds(sb, stripe)], acc_d.at[pl.ds(sb, stripe)])
        one16 = jnp.ones((16,), jnp.float32)
        for r in range(_CH):
            ones_v[r] = one16
        plsc.subcore_barrier()

        @pl.loop(0, K // _SEG)
        def _(g):
            segbase = wid * K + g * _SEG
            pltpu.sync_copy(src_hbm.at[pl.ds(segbase, _SEG)], src_v)
            pltpu.sync_copy(dst_hbm.at[pl.ds(segbase, _SEG)], dst_v)

            @pl.loop(0, _SEG)
            def _(m):
                base = (segbase + m) * _CH
                for j in range(_CH // 16):
                    eidx_v[pl.ds(16 * j, 16)] = (lax.iota(jnp.int32, 16)
                                                 + base + 16 * j)
                c_ef = pltpu.async_copy(ef_hbm.at[eidx_v], ef_v, sem2)
                c_h = pltpu.async_copy(h_hbm.at[src_v.at[m]], rows_v, sem)
                c_ef.wait()
                c_h.wait()
                s1 = pltpu.async_copy(rows_v, acc_h.at[dst_v.at[m]], sem3, add=True)
                s2 = pltpu.async_copy(ef_v, acc_e.at[dst_v.at[m]], sem3, add=True)
                s3 = pltpu.async_copy(ones_v, acc_d.at[dst_v.at[m]], sem3, add=True)
                s1.wait()
                s2.wait()
                s3.wait()

        plsc.subcore_barrier()
        ob = c * _NP + sb
        for t in range(stripe // _CH):
            pltpu.sync_copy(acc_h.at[pl.ds(sb + t * _CH, _CH)],
                            out_h.at[pl.ds(ob + t * _CH, _CH)])
        pltpu.sync_copy(acc_e.at[pl.ds(sb, stripe)], out_e.at[pl.ds(ob, stripe)])
        pltpu.sync_copy(acc_d.at[pl.ds(sb, stripe)], out_d.at[pl.ds(ob, stripe)])

    return sc_k


def _tc_body(hs_ref, ah0_ref, ah1_ref, ae0_ref, ae1_ref, ad0_ref, ad1_ref,
             wsT_ref, wn1T_ref, wn2T_ref, o_ref):
    deg = ad0_ref[:, 0:1] + ad1_ref[:, 0:1]
    inv = 1.0 / jnp.where(deg == 0.0, 1.0, deg)
    nm = (ah0_ref[...] + ah1_ref[...]) * inv
    em = (ae0_ref[...] + ae1_ref[...]) * inv
    hi = jax.lax.Precision.HIGHEST
    z = (jnp.dot(hs_ref[...], wsT_ref[...], precision=hi)
         + jnp.dot(nm, wn1T_ref[...], precision=hi)
         + jnp.dot(em, wn2T_ref[...], precision=hi))
    z = jnp.maximum(z, 0.0)
    nrm = jnp.sqrt(jnp.sum(z * z, axis=1, keepdims=True))
    o_ref[...] = z / jnp.where(nrm == 0.0, 1.0, nrm)


@functools.lru_cache(maxsize=None)
def _build_tc(N, DF, DE, DO, blk):
    g = N // blk
    ob = _NP // blk

    def spec(d, off):
        return pl.BlockSpec((blk, d), lambda i, o=off: (i + o, 0))

    full = lambda a, b: pl.BlockSpec((a, b), lambda i: (0, 0))
    return pl.pallas_call(
        _tc_body,
        grid=(g,),
        in_specs=[
            pl.BlockSpec((blk, DF), lambda i: (i, 0)),
            spec(DF, 0), spec(DF, ob),
            spec(DE, 0), spec(DE, ob),
            spec(DE, 0), spec(DE, ob),
            full(DF, DO), full(DF, DO), full(DE, DO),
        ],
        out_specs=pl.BlockSpec((blk, DO), lambda i: (i, 0)),
        out_shape=jax.ShapeDtypeStruct((N, DO), jnp.float32),
    )


def kernel(h_neigh, h_self, edge_index, edge_features, W_self, W_neigh):
    N, DF = h_neigh.shape
    E = edge_index.shape[1]
    DE = edge_features.shape[1]
    DO = W_self.shape[0]
    src = edge_index[0].reshape(E // _CH, _CH)
    dst = edge_index[1].reshape(E // _CH, _CH)
    z_f = jnp.zeros((_NP, DF), jnp.float32)
    z_e = jnp.zeros((_NP, DE), jnp.float32)
    out_h, out_e, out_d = _build_sc(N, E, DF, DE)(
        h_neigh, src, dst, edge_features, z_f, z_e)
    wsT = W_self.T
    wn1T = W_neigh[:, :DF].T
    wn2T = W_neigh[:, DF:].T
    return _build_tc(N, DF, DE, DO, 80)(
        h_self, out_h, out_h, out_e, out_e, out_d, out_d, wsT, wn1T, wn2T)


# speedup vs baseline: 5.4425x; 1.1918x over previous
import functools

import jax
import jax.numpy as jnp
from jax import lax
from jax.experimental import pallas as pl
from jax.experimental.pallas import tpu as pltpu
from jax.experimental.pallas import tpu_sc as plsc

_NC = 2
_NS = 16
_CH = 80
_SEG = 25
_NP = 10240


@functools.lru_cache(maxsize=None)
def _build_sc(N, E, DF, DE):
    R = E // _CH
    K = R // (_NC * _NS)
    assert R % (_NC * _NS) == 0
    stripe = _NP // _NS
    mesh = plsc.VectorSubcoreMesh(core_axis_name="c", subcore_axis_name="s",
                                  num_cores=_NC, num_subcores=_NS)

    @functools.partial(
        pl.kernel,
        out_type=(
            jax.ShapeDtypeStruct((_NC * _NP, DF), jnp.float32),
            jax.ShapeDtypeStruct((_NC * _NP, DE), jnp.float32),
            jax.ShapeDtypeStruct((_NC * _NP, DE), jnp.float32),
        ),
        mesh=mesh,
        compiler_params=pltpu.CompilerParams(use_tc_tiling_on_sc=False),
        scratch_types=[
            pltpu.VMEM_SHARED((_NP, DF), jnp.float32),
            pltpu.VMEM_SHARED((_NP, DE), jnp.float32),
            pltpu.VMEM_SHARED((_NP, DE), jnp.float32),
            pltpu.VMEM((_SEG, _CH), jnp.int32),
            pltpu.VMEM((_SEG, _CH), jnp.int32),
            pltpu.VMEM((_CH,), jnp.int32),
            pltpu.VMEM((_CH, DE), jnp.float32),
            pltpu.VMEM((_CH, DE), jnp.float32),
            pltpu.VMEM((_CH, DF), jnp.float32),
            pltpu.SemaphoreType.DMA,
            pltpu.SemaphoreType.DMA,
            pltpu.SemaphoreType.DMA,
        ],
    )
    def sc_k(h_hbm, ei_hbm, ef_hbm, z_f_hbm, z_e_hbm,
             out_h, out_e, out_d,
             acc_h, acc_e, acc_d, src_v, dst_v, eidx_v, ef_v, ones_v,
             rows_v, sem, sem2, sem3):
        c = lax.axis_index("c")
        s = lax.axis_index("s")
        wid = s * _NC + c
        sb = s * stripe
        for t in range(stripe // _CH):
            o = sb + t * _CH
            pltpu.sync_copy(z_f_hbm.at[pl.ds(o, _CH)], acc_h.at[pl.ds(o, _CH)])
        pltpu.sync_copy(z_e_hbm.at[pl.ds(sb, stripe)], acc_e.at[pl.ds(sb, stripe)])
        pltpu.sync_copy(z_e_hbm.at[pl.ds(sb, stripe)], acc_d.at[pl.ds(sb, stripe)])
        one16 = jnp.ones((16,), jnp.float32)
        for r in range(_CH):
            ones_v[r] = one16
        plsc.subcore_barrier()

        @pl.loop(0, K // _SEG)
        def _(g):
            segbase = wid * K + g * _SEG
            pltpu.sync_copy(ei_hbm.at[0, pl.ds(segbase, _SEG)], src_v)
            pltpu.sync_copy(ei_hbm.at[1, pl.ds(segbase, _SEG)], dst_v)

            @pl.loop(0, _SEG)
            def _(m):
                base = (segbase + m) * _CH
                for j in range(_CH // 16):
                    eidx_v[pl.ds(16 * j, 16)] = (lax.iota(jnp.int32, 16)
                                                 + base + 16 * j)
                c_ef = pltpu.async_copy(ef_hbm.at[eidx_v], ef_v, sem2)
                c_h = pltpu.async_copy(h_hbm.at[src_v.at[m]], rows_v, sem)
                c_ef.wait()
                c_h.wait()
                s1 = pltpu.async_copy(rows_v, acc_h.at[dst_v.at[m]], sem3, add=True)
                s2 = pltpu.async_copy(ef_v, acc_e.at[dst_v.at[m]], sem3, add=True)
                s3 = pltpu.async_copy(ones_v, acc_d.at[dst_v.at[m]], sem3, add=True)
                s1.wait()
                s2.wait()
                s3.wait()

        plsc.subcore_barrier()
        ob = c * _NP + sb
        for t in range(stripe // _CH):
            pltpu.sync_copy(acc_h.at[pl.ds(sb + t * _CH, _CH)],
                            out_h.at[pl.ds(ob + t * _CH, _CH)])
        pltpu.sync_copy(acc_e.at[pl.ds(sb, stripe)], out_e.at[pl.ds(ob, stripe)])
        pltpu.sync_copy(acc_d.at[pl.ds(sb, stripe)], out_d.at[pl.ds(ob, stripe)])

    return sc_k


def _tc_body(hs_ref, ah_ref, ae_ref, ad_ref,
             wsT_ref, wn1T_ref, wn2T_ref, o_ref):
    deg = ad_ref[0, :, 0:1] + ad_ref[1, :, 0:1]
    inv = 1.0 / jnp.where(deg == 0.0, 1.0, deg)
    nm = (ah_ref[0] + ah_ref[1]) * inv
    em = (ae_ref[0] + ae_ref[1]) * inv
    z = (jnp.dot(hs_ref[...], wsT_ref[...])
         + jnp.dot(nm, wn1T_ref[...])
         + jnp.dot(em, wn2T_ref[...]))
    z = jnp.maximum(z, 0.0)
    nrm = jnp.sqrt(jnp.sum(z * z, axis=1, keepdims=True))
    o_ref[...] = z / jnp.where(nrm == 0.0, 1.0, nrm)


@functools.lru_cache(maxsize=None)
def _build_tc(N, DF, DE, DO, blk):
    g = N // blk

    def spec(d):
        return pl.BlockSpec((2, blk, d), lambda i: (0, i, 0))

    full = lambda a, b: pl.BlockSpec((a, b), lambda i: (0, 0))
    return pl.pallas_call(
        _tc_body,
        grid=(g,),
        in_specs=[
            pl.BlockSpec((blk, DF), lambda i: (i, 0)),
            spec(DF), spec(DE), spec(DE),
            full(DF, DO), full(DF, DO), full(DE, DO),
        ],
        out_specs=pl.BlockSpec((blk, DO), lambda i: (i, 0)),
        out_shape=jax.ShapeDtypeStruct((N, DO), jnp.float32),
    )


def kernel(h_neigh, h_self, edge_index, edge_features, W_self, W_neigh):
    N, DF = h_neigh.shape
    E = edge_index.shape[1]
    DE = edge_features.shape[1]
    DO = W_self.shape[0]
    ei = edge_index.reshape(2, E // _CH, _CH)
    z_f = jnp.zeros((_NP, DF), jnp.float32)
    z_e = jnp.zeros((_NP, DE), jnp.float32)
    out_h, out_e, out_d = _build_sc(N, E, DF, DE)(
        h_neigh, ei, edge_features, z_f, z_e)
    wsT = W_self.T
    wn1T = W_neigh[:, :DF].T
    wn2T = W_neigh[:, DF:].T
    return _build_tc(N, DF, DE, DO, 1000)(
        h_self,
        out_h.reshape(2, _NP, DF),
        out_e.reshape(2, _NP, DE),
        out_d.reshape(2, _NP, DE),
        wsT, wn1T, wn2T)
